# baseline (speedup 1.0000x reference)
import functools

import jax
import jax.numpy as jnp
from jax.experimental import pallas as pl

_K = 5
_NEG = -3.0e38


def _topk_head_kernel(x_ref, w_ref, o_ref):
    x = x_ref[...]
    bb, p, s = x.shape
    ones = jnp.ones((s, 1), jnp.float32)
    acc = jnp.zeros((bb, p, 1), jnp.float32)
    need = jnp.full((bb, p, 1), float(_K), jnp.float32)
    for _ in range(_K):
        m = jnp.max(x, axis=-1, keepdims=True)
        geq = x >= m
        c = jax.lax.dot_general(
            geq.astype(jnp.float32), ones, (((2,), (0,)), ((), ())),
            preferred_element_type=jnp.float32,
        )
        take = jnp.minimum(c, jnp.maximum(need, 0.0))
        acc = acc + m * take
        need = need - c
        x = jnp.where(geq, _NEG, x)
    sim = acc[..., 0] * (1.0 / _K)
    o_ref[...] = jnp.dot(sim, w_ref[...], preferred_element_type=jnp.float32)


def kernel(prototype_activations, W):
    b, p = prototype_activations.shape[:2]
    s = prototype_activations.shape[2] * prototype_activations.shape[3]
    x = prototype_activations.reshape(b, p, s)
    wt = W.T
    c = W.shape[0]
    bb = 16
    grid = (b // bb,)
    out = pl.pallas_call(
        _topk_head_kernel,
        grid=grid,
        in_specs=[
            pl.BlockSpec((bb, p, s), lambda i: (i, 0, 0)),
            pl.BlockSpec((p, c), lambda i: (0, 0)),
        ],
        out_specs=pl.BlockSpec((bb, c), lambda i: (i, 0)),
        out_shape=jax.ShapeDtypeStruct((b, c), jnp.float32),
    )(x, wt)
    return out


# speedup vs baseline: 1.1138x; 1.1138x over previous
import functools

import jax
import jax.numpy as jnp
from jax.experimental import pallas as pl

_K = 5
_NEG = -3.0e38


def _topk_head_kernel(x_ref, w_ref, o_ref):
    x = x_ref[...]
    bb, p, s = x.shape
    ones = jnp.ones((s, 1), jnp.float32)
    acc = jnp.zeros((bb, p, 1), jnp.float32)
    need = jnp.full((bb, p, 1), float(_K), jnp.float32)
    for _ in range(_K):
        m = jnp.max(x, axis=-1, keepdims=True)
        geq = x >= m
        c = jnp.sum(geq.astype(jnp.float32), axis=-1, keepdims=True)
        take = jnp.minimum(c, jnp.maximum(need, 0.0))
        acc = acc + m * take
        need = need - c
        x = jnp.where(geq, _NEG, x)
    sim = acc[..., 0] * (1.0 / _K)
    o_ref[...] = jnp.dot(sim, w_ref[...], preferred_element_type=jnp.float32)


def kernel(prototype_activations, W):
    b, p = prototype_activations.shape[:2]
    s = prototype_activations.shape[2] * prototype_activations.shape[3]
    x = prototype_activations.reshape(b, p, s)
    wt = W.T
    c = W.shape[0]
    bb = 32
    grid = (b // bb,)
    out = pl.pallas_call(
        _topk_head_kernel,
        grid=grid,
        in_specs=[
            pl.BlockSpec((bb, p, s), lambda i: (i, 0, 0)),
            pl.BlockSpec((p, c), lambda i: (0, 0)),
        ],
        out_specs=pl.BlockSpec((bb, c), lambda i: (i, 0)),
        out_shape=jax.ShapeDtypeStruct((b, c), jnp.float32),
    )(x, wt)
    return out
